# Initial kernel scaffold; baseline (speedup 1.0000x reference)
import functools

import jax
import jax.numpy as jnp
from jax import lax
from jax.experimental import pallas as pl
from jax.experimental.pallas import tpu as pltpu
from jax.experimental.pallas import tpu_sc as plsc

_N = 320000
_D = 128
_C = 1000
_CPAD = 1024
_NC = 2
_NS = 16
_NW = _NC * _NS
_RPT = _N // _NW
_CHUNK = 80
_NCHUNK = _RPT // _CHUNK
_CROWS = _CPAD // _NS


def _phase1_body(x_hbm, lbl_hbm, zrow_hbm, zcnt_hbm, ones_hbm,
                 psum_hbm, psq_hbm, pcnt_hbm,
                 acc_sum, acc_sq, acc_cnt,
                 xbuf, sqbuf, lblbuf, obuf, zbuf, zcbuf):
    cid = lax.axis_index("c")
    sid = lax.axis_index("s")
    wid = sid * _NC + cid

    pltpu.sync_copy(zrow_hbm.at[pl.ds(0, _CROWS)], zbuf)
    pltpu.sync_copy(zcnt_hbm.at[pl.ds(0, _CROWS)], zcbuf)
    pltpu.sync_copy(ones_hbm, obuf)
    pltpu.sync_copy(zbuf, acc_sum.at[pl.ds(sid * _CROWS, _CROWS)])
    pltpu.sync_copy(zbuf, acc_sq.at[pl.ds(sid * _CROWS, _CROWS)])
    pltpu.sync_copy(zcbuf, acc_cnt.at[pl.ds(sid * _CROWS, _CROWS)])
    plsc.subcore_barrier()

    row_base = wid * _RPT

    def chunk_body(g, _):
        base = row_base + g * _CHUNK
        pltpu.sync_copy(x_hbm.at[pl.ds(base, _CHUNK)], xbuf)
        pltpu.sync_copy(lbl_hbm.at[pl.ds(base, _CHUNK)], lblbuf)

        def row_body(r, _):
            for cgrp in range(_D // 16):
                v = xbuf[r, pl.ds(cgrp * 16, 16)]
                sqbuf[r, pl.ds(cgrp * 16, 16)] = v * v
            return 0

        lax.fori_loop(0, _CHUNK, row_body, 0)

        pltpu.sync_copy(xbuf, acc_sum.at[lblbuf], add=True)
        pltpu.sync_copy(sqbuf, acc_sq.at[lblbuf], add=True)
        pltpu.sync_copy(obuf, acc_cnt.at[lblbuf], add=True)
        return 0

    lax.fori_loop(0, _NCHUNK, chunk_body, 0)
    plsc.subcore_barrier()

    rows = pl.ds(sid * _CROWS, _CROWS)
    pltpu.sync_copy(acc_sum.at[rows], zbuf)
    pltpu.sync_copy(zbuf, psum_hbm.at[cid, rows])
    pltpu.sync_copy(acc_sq.at[rows], zbuf)
    pltpu.sync_copy(zbuf, psq_hbm.at[cid, rows])
    pltpu.sync_copy(acc_cnt.at[rows], zcbuf)
    pltpu.sync_copy(zcbuf, pcnt_hbm.at[cid, rows])


def _phase1(x, labels, zrow, zcnt, ones):
    mesh = plsc.VectorSubcoreMesh(core_axis_name="c", subcore_axis_name="s")
    f32 = jnp.float32
    kern = functools.partial(
        pl.kernel,
        mesh=mesh,
        out_type=[
            jax.ShapeDtypeStruct((_NC, _CPAD, _D), f32),
            jax.ShapeDtypeStruct((_NC, _CPAD, _D), f32),
            jax.ShapeDtypeStruct((_NC, _CPAD, 16), f32),
        ],
        scratch_types=[
            pltpu.VMEM_SHARED((_CPAD, _D), f32),
            pltpu.VMEM_SHARED((_CPAD, _D), f32),
            pltpu.VMEM_SHARED((_CPAD, 16), f32),
            pltpu.VMEM((_CHUNK, _D), f32),
            pltpu.VMEM((_CHUNK, _D), f32),
            pltpu.VMEM((_CHUNK,), jnp.int32),
            pltpu.VMEM((_CHUNK, 16), f32),
            pltpu.VMEM((_CROWS, _D), f32),
            pltpu.VMEM((_CROWS, 16), f32),
        ],
    )(_phase1_body)
    return kern(x, labels, zrow, zcnt, ones)


def _phase2_body(ps_ref, pq_ref, pc_ref, rm_ref, rv_ref, cc_ref,
                 mean_ref, var_ref, cnt_ref):
    s = ps_ref[0, : _C, :] + ps_ref[1, : _C, :]
    q = pq_ref[0, : _C, :] + pq_ref[1, : _C, :]
    cnt16 = pc_ref[0, : _C, :] + pc_ref[1, : _C, :]
    counts = jnp.broadcast_to(cnt16[:, 0:1], (_C, _D))
    cc = cc_ref[...]
    rm = rm_ref[...]
    rv = rv_ref[...]
    total = cc + counts
    denom = jnp.maximum(total, 1.0)
    new_mean = (rm * cc + s) / denom
    m2 = q - 2.0 * new_mean * s + counts * new_mean * new_mean
    new_var = (rv * cc + m2) / denom
    mean_ref[...] = new_mean
    var_ref[...] = new_var
    cnt_ref[...] = (cc + counts).astype(jnp.int32)


def _phase2(ps, pq, pc, rm, rv, ccb):
    f32 = jnp.float32
    return pl.pallas_call(
        _phase2_body,
        out_shape=[
            jax.ShapeDtypeStruct((_C, _D), f32),
            jax.ShapeDtypeStruct((_C, _D), f32),
            jax.ShapeDtypeStruct((_C, _D), jnp.int32),
        ],
    )(ps, pq, pc, rm, rv, ccb)


def kernel(x, labels, running_mean, running_var, class_count):
    labels_i = labels.astype(jnp.int32)
    zrow = jnp.zeros((_CPAD, _D), jnp.float32)
    zcnt = jnp.zeros((_CPAD, 16), jnp.float32)
    ones = jnp.ones((_CHUNK, 16), jnp.float32)
    ps, pq, pc = _phase1(x, labels_i, zrow, zcnt, ones)
    ccb = jnp.broadcast_to(class_count.astype(jnp.float32), (_C, _D))
    new_mean, new_var, cnt_full = _phase2(ps, pq, pc, running_mean,
                                          running_var, ccb)
    return new_mean, new_var, cnt_full[:, :1]


# speedup vs baseline: 5.1745x; 5.1745x over previous
import functools

import jax
import jax.numpy as jnp
from jax import lax
from jax.experimental import pallas as pl
from jax.experimental.pallas import tpu as pltpu
from jax.experimental.pallas import tpu_sc as plsc

_N = 320000
_D = 128
_C = 1000
_CPAD = 1024
_NC = 2
_NS = 16
_NW = _NC * _NS
_RPT = _N // _NW
_CHUNK = 80
_NCHUNK = _RPT // _CHUNK
_CROWS = _CPAD // _NS


def _phase1_body(x_hbm, lbl_hbm, zrow_hbm, ones_hbm,
                 psum_hbm, psq_hbm, pcnt_hbm,
                 acc_sum, acc_sq, acc_cnt,
                 xbuf, sqbuf, lblbuf, obuf, zbuf):
    cid = lax.axis_index("c")
    sid = lax.axis_index("s")
    wid = sid * _NC + cid

    pltpu.sync_copy(zrow_hbm.at[pl.ds(0, _CROWS)], zbuf)
    pltpu.sync_copy(ones_hbm, obuf)
    pltpu.sync_copy(zbuf, acc_sum.at[pl.ds(sid * _CROWS, _CROWS)])
    pltpu.sync_copy(zbuf, acc_sq.at[pl.ds(sid * _CROWS, _CROWS)])
    pltpu.sync_copy(zbuf, acc_cnt.at[pl.ds(sid * _CROWS, _CROWS)])
    plsc.subcore_barrier()

    row_base = wid * _RPT

    def chunk_body(g, _):
        base = row_base + g * _CHUNK
        pltpu.sync_copy(x_hbm.at[pl.ds(base, _CHUNK)], xbuf)
        pltpu.sync_copy(lbl_hbm.at[pl.ds(base, _CHUNK)], lblbuf)

        def row_body(r, _):
            for cgrp in range(_D // 16):
                v = xbuf[r, pl.ds(cgrp * 16, 16)]
                sqbuf[r, pl.ds(cgrp * 16, 16)] = v * v
            return 0

        lax.fori_loop(0, _CHUNK, row_body, 0)

        pltpu.sync_copy(xbuf, acc_sum.at[lblbuf], add=True)
        pltpu.sync_copy(sqbuf, acc_sq.at[lblbuf], add=True)
        pltpu.sync_copy(obuf, acc_cnt.at[lblbuf], add=True)
        return 0

    lax.fori_loop(0, _NCHUNK, chunk_body, 0)
    plsc.subcore_barrier()

    rows = pl.ds(sid * _CROWS, _CROWS)
    pltpu.sync_copy(acc_sum.at[rows], zbuf)
    pltpu.sync_copy(zbuf, psum_hbm.at[cid, rows])
    pltpu.sync_copy(acc_sq.at[rows], zbuf)
    pltpu.sync_copy(zbuf, psq_hbm.at[cid, rows])
    pltpu.sync_copy(acc_cnt.at[rows], zbuf)
    pltpu.sync_copy(zbuf, pcnt_hbm.at[cid, rows])


def _phase1(x, labels, zrow, ones):
    mesh = plsc.VectorSubcoreMesh(core_axis_name="c", subcore_axis_name="s")
    f32 = jnp.float32
    kern = functools.partial(
        pl.kernel,
        mesh=mesh,
        out_type=[
            jax.ShapeDtypeStruct((_NC, _CPAD, _D), f32),
            jax.ShapeDtypeStruct((_NC, _CPAD, _D), f32),
            jax.ShapeDtypeStruct((_NC, _CPAD, _D), f32),
        ],
        scratch_types=[
            pltpu.VMEM_SHARED((_CPAD, _D), f32),
            pltpu.VMEM_SHARED((_CPAD, _D), f32),
            pltpu.VMEM_SHARED((_CPAD, _D), f32),
            pltpu.VMEM((_CHUNK, _D), f32),
            pltpu.VMEM((_CHUNK, _D), f32),
            pltpu.VMEM((_CHUNK,), jnp.int32),
            pltpu.VMEM((_CHUNK, _D), f32),
            pltpu.VMEM((_CROWS, _D), f32),
        ],
    )(_phase1_body)
    return kern(x, labels, zrow, ones)


def _phase2_body(ps_ref, pq_ref, pc_ref, rm_ref, rv_ref, cc_ref,
                 mean_ref, var_ref, cnt_ref):
    s = ps_ref[0, : _C, :] + ps_ref[1, : _C, :]
    q = pq_ref[0, : _C, :] + pq_ref[1, : _C, :]
    counts = jnp.broadcast_to(
        pc_ref[0, : _C, 0:1] + pc_ref[1, : _C, 0:1], (_C, _D))
    cc = cc_ref[...]
    rm = rm_ref[...]
    rv = rv_ref[...]
    total = cc + counts
    denom = jnp.maximum(total, 1.0)
    new_mean = (rm * cc + s) / denom
    m2 = q - 2.0 * new_mean * s + counts * new_mean * new_mean
    new_var = (rv * cc + m2) / denom
    mean_ref[...] = new_mean
    var_ref[...] = new_var
    cnt_ref[...] = (cc + counts).astype(jnp.int32)


def _phase2(ps, pq, pc, rm, rv, ccb):
    f32 = jnp.float32
    return pl.pallas_call(
        _phase2_body,
        out_shape=[
            jax.ShapeDtypeStruct((_C, _D), f32),
            jax.ShapeDtypeStruct((_C, _D), f32),
            jax.ShapeDtypeStruct((_C, _D), jnp.int32),
        ],
    )(ps, pq, pc, rm, rv, ccb)


def kernel(x, labels, running_mean, running_var, class_count):
    labels_i = labels.astype(jnp.int32)
    zrow = jnp.zeros((_CPAD, _D), jnp.float32)
    ones = jnp.ones((_CHUNK, _D), jnp.float32)
    ps, pq, pc = _phase1(x, labels_i, zrow, ones)
    ccb = jnp.broadcast_to(class_count.astype(jnp.float32), (_C, _D))
    new_mean, new_var, cnt_full = _phase2(ps, pq, pc, running_mean,
                                          running_var, ccb)
    return new_mean, new_var, cnt_full[:, :1]


# speedup vs baseline: 8.7449x; 1.6900x over previous
import functools

import jax
import jax.numpy as jnp
from jax import lax
from jax.experimental import pallas as pl
from jax.experimental.pallas import tpu as pltpu
from jax.experimental.pallas import tpu_sc as plsc

_N = 320000
_D = 128
_C = 1000
_CPAD = 1024
_NC = 2
_NS = 16
_NW = _NC * _NS
_RPT = _N // _NW
_CHUNK = 80
_NCHUNK = _RPT // _CHUNK
_CROWS = _CPAD // _NS


def _phase1_body(x_hbm, lbl_hbm, zrow_hbm, ones_hbm,
                 psum_hbm, psq_hbm, pcnt_hbm,
                 acc_sum, acc_sq, acc_cnt,
                 xb0, xb1, sq0, sq1, lb0, lb1, obuf, zbuf,
                 sx0, sx1, sl0, sl1):
    cid = lax.axis_index("c")
    sid = lax.axis_index("s")
    wid = sid * _NC + cid

    pltpu.sync_copy(zrow_hbm.at[pl.ds(0, _CROWS)], zbuf)
    pltpu.sync_copy(ones_hbm, obuf)
    pltpu.sync_copy(zbuf, acc_sum.at[pl.ds(sid * _CROWS, _CROWS)])
    pltpu.sync_copy(zbuf, acc_sq.at[pl.ds(sid * _CROWS, _CROWS)])
    pltpu.sync_copy(zbuf, acc_cnt.at[pl.ds(sid * _CROWS, _CROWS)])
    plsc.subcore_barrier()

    row_base = wid * _RPT
    bufs = ((xb0, sq0, lb0, sx0, sl0), (xb1, sq1, lb1, sx1, sl1))

    def start_fetch(g, xb, lb, sx, sl):
        base = row_base + g * _CHUNK
        pltpu.async_copy(x_hbm.at[pl.ds(base, _CHUNK)], xb, sx)
        pltpu.async_copy(lbl_hbm.at[pl.ds(base, _CHUNK)], lb, sl)

    def process(g, xb, sq, lb, sx, sl):
        pltpu.make_async_copy(
            x_hbm.at[pl.ds(row_base, _CHUNK)], xb, sx).wait()
        pltpu.make_async_copy(
            lbl_hbm.at[pl.ds(row_base, _CHUNK)], lb, sl).wait()

        def row_body(r, _):
            for cgrp in range(_D // 16):
                v = xb[r, pl.ds(cgrp * 16, 16)]
                sq[r, pl.ds(cgrp * 16, 16)] = v * v
            return 0

        lax.fori_loop(0, _CHUNK, row_body, 0)

        pltpu.sync_copy(xb, acc_sum.at[lb], add=True)
        pltpu.sync_copy(sq, acc_sq.at[lb], add=True)
        pltpu.sync_copy(obuf, acc_cnt.at[lb], add=True)

    start_fetch(0, xb0, lb0, sx0, sl0)
    start_fetch(1, xb1, lb1, sx1, sl1)

    def pair_body(i, _):
        for b in range(2):
            xb, sq, lb, sx, sl = bufs[b]
            g = 2 * i + b
            process(g, xb, sq, lb, sx, sl)

            @pl.when(g + 2 < _NCHUNK)
            def _():
                start_fetch(g + 2, xb, lb, sx, sl)
        return 0

    lax.fori_loop(0, (_NCHUNK - 1) // 2, pair_body, 0)
    process(_NCHUNK - 1, xb0, sq0, lb0, sx0, sl0)
    plsc.subcore_barrier()

    rows = pl.ds(sid * _CROWS, _CROWS)
    pltpu.sync_copy(acc_sum.at[rows], zbuf)
    pltpu.sync_copy(zbuf, psum_hbm.at[cid, rows])
    pltpu.sync_copy(acc_sq.at[rows], zbuf)
    pltpu.sync_copy(zbuf, psq_hbm.at[cid, rows])
    pltpu.sync_copy(acc_cnt.at[rows], zbuf)
    pltpu.sync_copy(zbuf, pcnt_hbm.at[cid, rows])


def _phase1(x, labels, zrow, ones):
    mesh = plsc.VectorSubcoreMesh(core_axis_name="c", subcore_axis_name="s")
    f32 = jnp.float32
    kern = functools.partial(
        pl.kernel,
        mesh=mesh,
        out_type=[
            jax.ShapeDtypeStruct((_NC, _CPAD, _D), f32),
            jax.ShapeDtypeStruct((_NC, _CPAD, _D), f32),
            jax.ShapeDtypeStruct((_NC, _CPAD, _D), f32),
        ],
        scratch_types=[
            pltpu.VMEM_SHARED((_CPAD, _D), f32),
            pltpu.VMEM_SHARED((_CPAD, _D), f32),
            pltpu.VMEM_SHARED((_CPAD, _D), f32),
            pltpu.VMEM((_CHUNK, _D), f32),
            pltpu.VMEM((_CHUNK, _D), f32),
            pltpu.VMEM((_CHUNK, _D), f32),
            pltpu.VMEM((_CHUNK, _D), f32),
            pltpu.VMEM((_CHUNK,), jnp.int32),
            pltpu.VMEM((_CHUNK,), jnp.int32),
            pltpu.VMEM((_CHUNK, _D), f32),
            pltpu.VMEM((_CROWS, _D), f32),
            pltpu.SemaphoreType.DMA,
            pltpu.SemaphoreType.DMA,
            pltpu.SemaphoreType.DMA,
            pltpu.SemaphoreType.DMA,
        ],
    )(_phase1_body)
    return kern(x, labels, zrow, ones)


def _phase2_body(ps_ref, pq_ref, pc_ref, rm_ref, rv_ref, cc_ref,
                 mean_ref, var_ref, cnt_ref):
    s = ps_ref[0, : _C, :] + ps_ref[1, : _C, :]
    q = pq_ref[0, : _C, :] + pq_ref[1, : _C, :]
    counts = jnp.broadcast_to(
        pc_ref[0, : _C, 0:1] + pc_ref[1, : _C, 0:1], (_C, _D))
    cc = cc_ref[...]
    rm = rm_ref[...]
    rv = rv_ref[...]
    total = cc + counts
    denom = jnp.maximum(total, 1.0)
    new_mean = (rm * cc + s) / denom
    m2 = q - 2.0 * new_mean * s + counts * new_mean * new_mean
    new_var = (rv * cc + m2) / denom
    mean_ref[...] = new_mean
    var_ref[...] = new_var
    cnt_ref[...] = (cc + counts).astype(jnp.int32)


def _phase2(ps, pq, pc, rm, rv, ccb):
    f32 = jnp.float32
    return pl.pallas_call(
        _phase2_body,
        out_shape=[
            jax.ShapeDtypeStruct((_C, _D), f32),
            jax.ShapeDtypeStruct((_C, _D), f32),
            jax.ShapeDtypeStruct((_C, _D), jnp.int32),
        ],
    )(ps, pq, pc, rm, rv, ccb)


def kernel(x, labels, running_mean, running_var, class_count):
    labels_i = labels.astype(jnp.int32)
    zrow = jnp.zeros((_CPAD, _D), jnp.float32)
    ones = jnp.ones((_CHUNK, _D), jnp.float32)
    ps, pq, pc = _phase1(x, labels_i, zrow, ones)
    ccb = jnp.broadcast_to(class_count.astype(jnp.float32), (_C, _D))
    new_mean, new_var, cnt_full = _phase2(ps, pq, pc, running_mean,
                                          running_var, ccb)
    return new_mean, new_var, cnt_full[:, :1]
